# baseline (speedup 1.0000x reference)
import jax
import jax.numpy as jnp
from jax import lax
from jax.experimental import pallas as pl
from jax.experimental.pallas import tpu as pltpu
from jax.experimental.pallas import tpu_sc as plsc

VOCAB = 1000000
D = 64
B = 16384
L = 20

NC = 2
NS = 16
LANES = 16
NW = NC * NS
BPW = B // NW
C = 32
NCH = BPW // C
IDX_ROWS = BPW * L // 128
ROWS_PER_CHUNK = C * L
GATHERS = ROWS_PER_CHUNK // 128


def _newton_sqrt(x):
    xs = jnp.maximum(x, jnp.float32(1e-30))
    i = plsc.bitcast(xs, jnp.int32)
    i = jnp.int32(0x5F3759DF) - (i >> 1)
    y = plsc.bitcast(i, jnp.float32)
    half = jnp.float32(0.5)
    threehalf = jnp.float32(1.5)
    for _ in range(3):
        y = y * (threehalf - half * xs * y * y)
    return xs * y


def _body(w_hbm, ctxidx_hbm, ctxt_hbm, tgt_hbm, out_hbm,
          idx_v, ctxt_v, tgt_v, w0_v, rows_v, trow_v,
          n0_v, cacc_v, red_v, stage_v, shared_v,
          sem_rows, sem_tgt):
    cid = lax.axis_index("c")
    sid = lax.axis_index("s")
    w = sid * NC + cid

    pltpu.sync_copy(ctxidx_hbm.at[pl.ds(w * IDX_ROWS, IDX_ROWS)], idx_v)
    pltpu.sync_copy(ctxt_hbm.at[w], ctxt_v)
    pltpu.sync_copy(tgt_hbm.at[w], tgt_v)
    pltpu.sync_copy(w_hbm.at[0], w0_v)

    eps = jnp.float32(1e-8)

    def chunk_body(c, cos_sum):
        for j in range(GATHERS):
            pltpu.async_copy(
                w_hbm.at[idx_v.at[c * GATHERS + j]],
                rows_v.at[pl.ds(j * 128, 128)], sem_rows)
        pltpu.async_copy(
            w_hbm.at[tgt_v.at[pl.ds(c * C, C)]], trow_v, sem_tgt)
        for j in range(GATHERS):
            pltpu.make_async_copy(
                w_hbm.at[idx_v.at[c * GATHERS + j]],
                rows_v.at[pl.ds(j * 128, 128)], sem_rows).wait()
        pltpu.make_async_copy(
            w_hbm.at[tgt_v.at[pl.ds(c * C, C)]], trow_v, sem_tgt).wait()

        for g in range(C // LANES):
            cnt = jnp.zeros((LANES,), jnp.int32)
            for l in range(L):
                ids = ctxt_v[l, pl.ds(c * C + g * LANES, LANES)]
                cnt = cnt + jnp.where(ids == jnp.int32(0),
                                      jnp.int32(1), jnp.int32(0))
            n0_v[pl.ds(g * LANES, LANES)] = cnt.astype(jnp.float32)

        def ebody(e, acc_cos):
            acc = [rows_v[e * L, pl.ds(16 * d, 16)] for d in range(4)]
            for l in range(1, L):
                for d in range(4):
                    acc[d] = acc[d] + rows_v[e * L + l, pl.ds(16 * d, 16)]
            n0 = jnp.full((LANES,), n0_v[pl.ds(e, LANES)][0], jnp.float32)
            inv = jnp.float32(1.0) / (jnp.float32(L) - n0 + jnp.float32(1e-6))
            pooled = [(acc[d] - n0 * w0_v[pl.ds(16 * d, 16)]) * inv
                      for d in range(4)]
            tg = [trow_v[e, pl.ds(16 * d, 16)] for d in range(4)]
            dv = pooled[0] * tg[0]
            av = pooled[0] * pooled[0]
            bv = tg[0] * tg[0]
            for d in range(1, 4):
                dv = dv + pooled[d] * tg[d]
                av = av + pooled[d] * pooled[d]
                bv = bv + tg[d] * tg[d]
            dots = jnp.full((LANES,), jnp.sum(dv), jnp.float32)
            na2 = jnp.full((LANES,), jnp.sum(av), jnp.float32)
            nb2 = jnp.full((LANES,), jnp.sum(bv), jnp.float32)
            na = _newton_sqrt(na2)
            nb = _newton_sqrt(nb2)
            cos = dots / (jnp.maximum(na, eps) * jnp.maximum(nb, eps))
            return acc_cos + cos

        return lax.fori_loop(0, C, ebody, cos_sum)

    total = lax.fori_loop(0, NCH, chunk_body,
                          jnp.zeros((LANES,), jnp.float32))

    cacc_v[...] = total
    pltpu.sync_copy(cacc_v, shared_v.at[sid])
    plsc.subcore_barrier()

    @pl.when(sid == 0)
    def _():
        pltpu.sync_copy(shared_v, red_v)
        tot = red_v[0, pl.ds(0, LANES)]
        for s in range(1, NS):
            tot = tot + red_v[s, pl.ds(0, LANES)]
        stage_v[...] = -tot / jnp.float32(B)
        pltpu.sync_copy(stage_v, out_hbm.at[cid])


@jax.jit
def _cbow_loss(ctxidx2d, ctxt, tgt2d, W):
    mesh = plsc.VectorSubcoreMesh(
        core_axis_name="c", subcore_axis_name="s",
        num_cores=NC, num_subcores=NS)
    partial = pl.kernel(
        _body,
        out_type=jax.ShapeDtypeStruct((NC, LANES), jnp.float32),
        mesh=mesh,
        compiler_params=pltpu.CompilerParams(
            needs_layout_passes=False, use_tc_tiling_on_sc=False),
        scratch_types=[
            pltpu.VMEM((IDX_ROWS, 128), jnp.int32),
            pltpu.VMEM((L, BPW), jnp.int32),
            pltpu.VMEM((BPW,), jnp.int32),
            pltpu.VMEM((D,), jnp.float32),
            pltpu.VMEM((ROWS_PER_CHUNK, D), jnp.float32),
            pltpu.VMEM((C, D), jnp.float32),
            pltpu.VMEM((C + LANES,), jnp.float32),
            pltpu.VMEM((LANES,), jnp.float32),
            pltpu.VMEM((NS, LANES), jnp.float32),
            pltpu.VMEM((LANES,), jnp.float32),
            pltpu.VMEM_SHARED((NS, LANES), jnp.float32),
            pltpu.SemaphoreType.DMA,
            pltpu.SemaphoreType.DMA,
        ],
    )(W, ctxidx2d, ctxt, tgt2d)
    return partial[0, 0] + partial[1, 0]


def kernel(contexts, targets, W):
    ctxidx2d = contexts.reshape(B * L // 128, 128)
    ctxt = contexts.reshape(NW, BPW, L).transpose(0, 2, 1)
    tgt2d = targets.reshape(NW, BPW)
    return _cbow_loss(ctxidx2d, ctxt, tgt2d, W)


# speedup vs baseline: 1.0414x; 1.0414x over previous
import jax
import jax.numpy as jnp
from jax import lax
from jax.experimental import pallas as pl
from jax.experimental.pallas import tpu as pltpu
from jax.experimental.pallas import tpu_sc as plsc

VOCAB = 1000000
D = 64
B = 16384
L = 20

NC = 2
NS = 16
LANES = 16
NW = NC * NS
BPW = B // NW
C = 32
NCH = BPW // C
IDX_ROWS = BPW * L // 128
ROWS_PER_CHUNK = C * L
GATHERS = ROWS_PER_CHUNK // 128


def _newton_sqrt(x):
    xs = jnp.maximum(x, jnp.float32(1e-30))
    i = plsc.bitcast(xs, jnp.int32)
    i = jnp.int32(0x5F3759DF) - (i >> 1)
    y = plsc.bitcast(i, jnp.float32)
    half = jnp.float32(0.5)
    threehalf = jnp.float32(1.5)
    for _ in range(3):
        y = y * (threehalf - half * xs * y * y)
    return xs * y


def _body(w_hbm, ctxidx_hbm, ctxt_hbm, tgt_hbm, out_hbm,
          idx_v, ctxt_v, tgt_v, w0_v, rows_v, trow_v,
          n0_v, cacc_v, red_v, stage_v, shared_v,
          sem_rows, sem_tgt):
    cid = lax.axis_index("c")
    sid = lax.axis_index("s")
    w = sid * NC + cid

    pltpu.sync_copy(ctxidx_hbm.at[pl.ds(w * IDX_ROWS, IDX_ROWS)], idx_v)
    pltpu.sync_copy(ctxt_hbm.at[w], ctxt_v)
    pltpu.sync_copy(tgt_hbm.at[w], tgt_v)
    pltpu.sync_copy(w_hbm.at[0], w0_v)

    eps = jnp.float32(1e-8)

    def issue(c, p):
        for j in range(GATHERS):
            pltpu.async_copy(
                w_hbm.at[idx_v.at[c * GATHERS + j]],
                rows_v.at[p, pl.ds(j * 128, 128)], sem_rows.at[p])
        pltpu.async_copy(
            w_hbm.at[tgt_v.at[pl.ds(c * C, C)]], trow_v.at[p],
            sem_tgt.at[p])

    def drain(c, p):
        for j in range(GATHERS):
            pltpu.make_async_copy(
                w_hbm.at[idx_v.at[c * GATHERS + j]],
                rows_v.at[p, pl.ds(j * 128, 128)], sem_rows.at[p]).wait()
        pltpu.make_async_copy(
            w_hbm.at[tgt_v.at[pl.ds(c * C, C)]], trow_v.at[p],
            sem_tgt.at[p]).wait()

    issue(0, 0)

    def chunk_body(c, cos_sum):
        p = lax.rem(c, 2)

        @pl.when(c + 1 < NCH)
        def _():
            issue(c + 1, 1 - p)

        drain(c, p)

        for g in range(C // LANES):
            cnt = jnp.zeros((LANES,), jnp.int32)
            for l in range(L):
                ids = ctxt_v[l, pl.ds(c * C + g * LANES, LANES)]
                cnt = cnt + jnp.where(ids == jnp.int32(0),
                                      jnp.int32(1), jnp.int32(0))
            n0_v[pl.ds(g * LANES, LANES)] = cnt.astype(jnp.float32)

        def ebody(e, acc_cos):
            acc = [rows_v[p, e * L, pl.ds(16 * d, 16)] for d in range(4)]
            for l in range(1, L):
                for d in range(4):
                    acc[d] = acc[d] + rows_v[p, e * L + l, pl.ds(16 * d, 16)]
            n0 = jnp.full((LANES,), n0_v[pl.ds(e, LANES)][0], jnp.float32)
            inv = jnp.float32(1.0) / (jnp.float32(L) - n0 + jnp.float32(1e-6))
            pooled = [(acc[d] - n0 * w0_v[pl.ds(16 * d, 16)]) * inv
                      for d in range(4)]
            tg = [trow_v[p, e, pl.ds(16 * d, 16)] for d in range(4)]
            dv = pooled[0] * tg[0]
            av = pooled[0] * pooled[0]
            bv = tg[0] * tg[0]
            for d in range(1, 4):
                dv = dv + pooled[d] * tg[d]
                av = av + pooled[d] * pooled[d]
                bv = bv + tg[d] * tg[d]
            dots = jnp.full((LANES,), jnp.sum(dv), jnp.float32)
            na2 = jnp.full((LANES,), jnp.sum(av), jnp.float32)
            nb2 = jnp.full((LANES,), jnp.sum(bv), jnp.float32)
            na = _newton_sqrt(na2)
            nb = _newton_sqrt(nb2)
            cos = dots / (jnp.maximum(na, eps) * jnp.maximum(nb, eps))
            return acc_cos + cos

        return lax.fori_loop(0, C, ebody, cos_sum)

    total = lax.fori_loop(0, NCH, chunk_body,
                          jnp.zeros((LANES,), jnp.float32))

    cacc_v[...] = total
    pltpu.sync_copy(cacc_v, shared_v.at[sid])
    plsc.subcore_barrier()

    @pl.when(sid == 0)
    def _():
        pltpu.sync_copy(shared_v, red_v)
        tot = red_v[0, pl.ds(0, LANES)]
        for s in range(1, NS):
            tot = tot + red_v[s, pl.ds(0, LANES)]
        stage_v[...] = -tot / jnp.float32(B)
        pltpu.sync_copy(stage_v, out_hbm.at[cid])


@jax.jit
def _cbow_loss(ctxidx2d, ctxt, tgt2d, W):
    mesh = plsc.VectorSubcoreMesh(
        core_axis_name="c", subcore_axis_name="s",
        num_cores=NC, num_subcores=NS)
    partial = pl.kernel(
        _body,
        out_type=jax.ShapeDtypeStruct((NC, LANES), jnp.float32),
        mesh=mesh,
        compiler_params=pltpu.CompilerParams(
            needs_layout_passes=False, use_tc_tiling_on_sc=False),
        scratch_types=[
            pltpu.VMEM((IDX_ROWS, 128), jnp.int32),
            pltpu.VMEM((L, BPW), jnp.int32),
            pltpu.VMEM((BPW,), jnp.int32),
            pltpu.VMEM((D,), jnp.float32),
            pltpu.VMEM((2, ROWS_PER_CHUNK, D), jnp.float32),
            pltpu.VMEM((2, C, D), jnp.float32),
            pltpu.VMEM((C + LANES,), jnp.float32),
            pltpu.VMEM((LANES,), jnp.float32),
            pltpu.VMEM((NS, LANES), jnp.float32),
            pltpu.VMEM((LANES,), jnp.float32),
            pltpu.VMEM_SHARED((NS, LANES), jnp.float32),
            pltpu.SemaphoreType.DMA((2,)),
            pltpu.SemaphoreType.DMA((2,)),
        ],
    )(W, ctxidx2d, ctxt, tgt2d)
    return partial[0, 0] + partial[1, 0]


def kernel(contexts, targets, W):
    ctxidx2d = contexts.reshape(B * L // 128, 128)
    ctxt = contexts.reshape(NW, BPW, L).transpose(0, 2, 1)
    tgt2d = targets.reshape(NW, BPW)
    return _cbow_loss(ctxidx2d, ctxt, tgt2d, W)


# speedup vs baseline: 1.0440x; 1.0025x over previous
import jax
import jax.numpy as jnp
from jax import lax
from jax.experimental import pallas as pl
from jax.experimental.pallas import tpu as pltpu
from jax.experimental.pallas import tpu_sc as plsc

VOCAB = 1000000
D = 64
B = 16384
L = 20

NC = 2
NS = 16
LANES = 16
NW = NC * NS
BPW = B // NW
C = 32
NCH = BPW // C
IDX_LEN = BPW * L
ROWS_PER_CHUNK = C * L
GATHERS = ROWS_PER_CHUNK // 128


def _newton_sqrt(x):
    xs = jnp.maximum(x, jnp.float32(1e-30))
    i = plsc.bitcast(xs, jnp.int32)
    i = jnp.int32(0x5F3759DF) - (i >> 1)
    y = plsc.bitcast(i, jnp.float32)
    half = jnp.float32(0.5)
    threehalf = jnp.float32(1.5)
    for _ in range(3):
        y = y * (threehalf - half * xs * y * y)
    return xs * y


def _body(w_hbm, ctxt_hbm, tgt_hbm, out_hbm,
          idx_v, ctxt_v, tgt_v, w0_v, rows_v, trow_v,
          n0_v, cacc_v, red_v, stage_v, shared_v,
          sem_rows, sem_tgt):
    cid = lax.axis_index("c")
    sid = lax.axis_index("s")
    w = sid * NC + cid

    pltpu.sync_copy(ctxt_hbm.at[:, pl.ds(w * BPW, BPW)], ctxt_v)
    pltpu.sync_copy(tgt_hbm.at[pl.ds(w * BPW, BPW)], tgt_v)
    pltpu.sync_copy(w_hbm.at[0], w0_v)

    lane = jnp.arange(LANES, dtype=jnp.int32)

    def tbody(g, carry):
        base = (g * LANES + lane) * L
        for l in range(L):
            ids = ctxt_v[l, pl.ds(g * LANES, LANES)]
            plsc.store_scatter(idx_v, [base + l], ids)
        return carry

    lax.fori_loop(0, BPW // LANES, tbody, 0)

    eps = jnp.float32(1e-8)

    def issue(c, p):
        for j in range(GATHERS):
            pltpu.async_copy(
                w_hbm.at[idx_v.at[pl.ds(c * ROWS_PER_CHUNK + j * 128, 128)]],
                rows_v.at[p, pl.ds(j * 128, 128)], sem_rows.at[p])
        pltpu.async_copy(
            w_hbm.at[tgt_v.at[pl.ds(c * C, C)]], trow_v.at[p],
            sem_tgt.at[p])

    def drain(c, p):
        for j in range(GATHERS):
            pltpu.make_async_copy(
                w_hbm.at[idx_v.at[pl.ds(c * ROWS_PER_CHUNK + j * 128, 128)]],
                rows_v.at[p, pl.ds(j * 128, 128)], sem_rows.at[p]).wait()
        pltpu.make_async_copy(
            w_hbm.at[tgt_v.at[pl.ds(c * C, C)]], trow_v.at[p],
            sem_tgt.at[p]).wait()

    issue(0, 0)

    def chunk_body(c, cos_sum):
        p = lax.rem(c, 2)

        @pl.when(c + 1 < NCH)
        def _():
            issue(c + 1, 1 - p)

        drain(c, p)

        for g in range(C // LANES):
            cnt = jnp.zeros((LANES,), jnp.int32)
            for l in range(L):
                ids = ctxt_v[l, pl.ds(c * C + g * LANES, LANES)]
                cnt = cnt + jnp.where(ids == jnp.int32(0),
                                      jnp.int32(1), jnp.int32(0))
            n0_v[pl.ds(g * LANES, LANES)] = cnt.astype(jnp.float32)

        def ebody(e, acc_cos):
            acc = [rows_v[p, e * L, pl.ds(16 * d, 16)] for d in range(4)]
            for l in range(1, L):
                for d in range(4):
                    acc[d] = acc[d] + rows_v[p, e * L + l, pl.ds(16 * d, 16)]
            n0 = jnp.full((LANES,), n0_v[pl.ds(e, LANES)][0], jnp.float32)
            inv = jnp.float32(1.0) / (jnp.float32(L) - n0 + jnp.float32(1e-6))
            pooled = [(acc[d] - n0 * w0_v[pl.ds(16 * d, 16)]) * inv
                      for d in range(4)]
            tg = [trow_v[p, e, pl.ds(16 * d, 16)] for d in range(4)]
            dv = pooled[0] * tg[0]
            av = pooled[0] * pooled[0]
            bv = tg[0] * tg[0]
            for d in range(1, 4):
                dv = dv + pooled[d] * tg[d]
                av = av + pooled[d] * pooled[d]
                bv = bv + tg[d] * tg[d]
            dots = jnp.full((LANES,), jnp.sum(dv), jnp.float32)
            na2 = jnp.full((LANES,), jnp.sum(av), jnp.float32)
            nb2 = jnp.full((LANES,), jnp.sum(bv), jnp.float32)
            na = _newton_sqrt(na2)
            nb = _newton_sqrt(nb2)
            cos = dots / (jnp.maximum(na, eps) * jnp.maximum(nb, eps))
            return acc_cos + cos

        return lax.fori_loop(0, C, ebody, cos_sum)

    total = lax.fori_loop(0, NCH, chunk_body,
                          jnp.zeros((LANES,), jnp.float32))

    cacc_v[...] = total
    pltpu.sync_copy(cacc_v, shared_v.at[sid])
    plsc.subcore_barrier()

    @pl.when(sid == 0)
    def _():
        pltpu.sync_copy(shared_v, red_v)
        tot = red_v[0, pl.ds(0, LANES)]
        for s in range(1, NS):
            tot = tot + red_v[s, pl.ds(0, LANES)]
        stage_v[...] = -tot / jnp.float32(B)
        pltpu.sync_copy(stage_v, out_hbm.at[cid])


@jax.jit
def _cbow_loss(ctxt, targets, W):
    mesh = plsc.VectorSubcoreMesh(
        core_axis_name="c", subcore_axis_name="s",
        num_cores=NC, num_subcores=NS)
    partial = pl.kernel(
        _body,
        out_type=jax.ShapeDtypeStruct((NC, LANES), jnp.float32),
        mesh=mesh,
        compiler_params=pltpu.CompilerParams(
            needs_layout_passes=False, use_tc_tiling_on_sc=False),
        scratch_types=[
            pltpu.VMEM((IDX_LEN,), jnp.int32),
            pltpu.VMEM((L, BPW), jnp.int32),
            pltpu.VMEM((BPW,), jnp.int32),
            pltpu.VMEM((D,), jnp.float32),
            pltpu.VMEM((2, ROWS_PER_CHUNK, D), jnp.float32),
            pltpu.VMEM((2, C, D), jnp.float32),
            pltpu.VMEM((C + LANES,), jnp.float32),
            pltpu.VMEM((LANES,), jnp.float32),
            pltpu.VMEM((NS, LANES), jnp.float32),
            pltpu.VMEM((LANES,), jnp.float32),
            pltpu.VMEM_SHARED((NS, LANES), jnp.float32),
            pltpu.SemaphoreType.DMA((2,)),
            pltpu.SemaphoreType.DMA((2,)),
        ],
    )(W, ctxt, targets)
    return partial[0, 0] + partial[1, 0]


def kernel(contexts, targets, W):
    return _cbow_loss(contexts.T, targets, W)
